# baseline (speedup 1.0000x reference)
import functools

import jax
import jax.numpy as jnp
from jax import lax
from jax.experimental import pallas as pl
from jax.experimental.pallas import tpu as pltpu
from jax.experimental.pallas import tpu_sc as plsc

_LANES = 128
_RB = 4096
_R_SC = 4096
_NW = 32
_B = 512
_F = 48


def _sc_shard_body(rpt, x_hbm, cs_hbm, rul_hbm, oc_hbm, out_hbm, x_v, cs_v,
                   ft_v, acc_v, idx_v, oc_v):
    wid = lax.axis_index("s") * 2 + lax.axis_index("c")
    pltpu.sync_copy(x_hbm, x_v)
    pltpu.sync_copy(cs_hbm, cs_v)
    pltpu.sync_copy(rul_hbm.at[wid], idx_v)
    pltpu.sync_copy(oc_hbm, oc_v)

    for c in range(42):
        v = c // 7
        cvec = cs_v[pl.ds((c // 16) * 16, 16)]
        svec = cs_v[pl.ds(48 + (c // 16) * 16, 16)]
        s2vec = 2.0 * svec * svec
        cc = cvec[c % 16]
        s2 = s2vec[c % 16]

        def fbody(bc, carry, v=v, c=c, cc=cc, s2=s2):
            d = x_v[pl.ds(v * _B + bc * 16, 16)] - cc
            ft_v[pl.ds(c * _B + bc * 16, 16)] = jnp.exp(-(d * d) / s2)
            return carry

        lax.fori_loop(0, _B // 16, fbody, 0)

    z16 = jnp.zeros((16,), jnp.float32)
    for k in range(3 * _B // 16):
        acc_v[pl.ds(k * 16, 16)] = z16

    def rbody(p, carry):
        iv = idx_v[pl.ds(p * 16, 16)]
        for half in range(2):
            o = half * 8
            ow0 = oc_v[pl.ds(iv[o + 6], 16)][0]
            ow1 = oc_v[pl.ds(iv[o + 7], 16)][0]
            for bc in range(_B // 16):
                base = bc * 16
                r0 = ft_v[pl.ds(iv[o + 0] * _B + base, 16)]
                r1 = ft_v[pl.ds(iv[o + 1] * _B + base, 16)]
                r2 = ft_v[pl.ds(iv[o + 2] * _B + base, 16)]
                r3 = ft_v[pl.ds(iv[o + 3] * _B + base, 16)]
                r4 = ft_v[pl.ds(iv[o + 4] * _B + base, 16)]
                r5 = ft_v[pl.ds(iv[o + 5] * _B + base, 16)]
                w = jnp.minimum(jnp.minimum(jnp.minimum(r0, r1),
                                            jnp.minimum(r2, r3)),
                                jnp.minimum(r4, r5))
                plsc.addupdate(acc_v.at[pl.ds(0 * _B + base, 16)], w * ow0)
                plsc.addupdate(acc_v.at[pl.ds(1 * _B + base, 16)], w * ow1)
                plsc.addupdate(acc_v.at[pl.ds(2 * _B + base, 16)], w)
        return carry

    lax.fori_loop(0, rpt // 2, rbody, 0)
    for k in range(3 * _B // 16, 8 * _B // 16):
        acc_v[pl.ds(k * 16, 16)] = z16
    pltpu.sync_copy(acc_v, out_hbm.at[wid])


def _sc_shard(x_flat, cs_flat, rules_sc, oc_pad):
    rpt = _R_SC // _NW
    mesh = plsc.VectorSubcoreMesh(core_axis_name="c", subcore_axis_name="s")
    kern = functools.partial(
        pl.kernel,
        out_type=jax.ShapeDtypeStruct((_NW, 8 * _B), jnp.float32),
        mesh=mesh,
        scratch_types=[
            pltpu.VMEM((8 * _B,), jnp.float32),
            pltpu.VMEM((96,), jnp.float32),
            pltpu.VMEM((_F * _B,), jnp.float32),
            pltpu.VMEM((8 * _B,), jnp.float32),
            pltpu.VMEM((rpt * 8,), jnp.int32),
            pltpu.VMEM((40,), jnp.float32),
        ],
    )(functools.partial(_sc_shard_body, rpt))
    return kern(x_flat, cs_flat, rules_sc, oc_pad)


def _tc_shard_body(nvar, xrep_ref, aux_ref, rules_ref, orules_ref, acc_out_ref,
                   acc_ref):
    i = pl.program_id(0)
    nb = pl.num_programs(0)
    rb = rules_ref.shape[1]

    @pl.when(i == 0)
    def _init():
        acc_ref[...] = jnp.zeros_like(acc_ref)

    c = aux_ref[0:1, :]
    s = aux_ref[1:2, :]
    d = xrep_ref[...] - c
    fuzz = jnp.exp(-(d * d) / (2.0 * s * s))

    iota = lax.broadcasted_iota(jnp.int32, (_LANES, rb), 0)

    wmin = None
    for v in range(nvar):
        oh = (iota == rules_ref[v:v + 1, :]).astype(jnp.float32)
        wv = jnp.dot(fuzz, oh, preferred_element_type=jnp.float32)
        wmin = wv if wmin is None else jnp.minimum(wmin, wv)

    oc = aux_ref[...]
    oh0 = (iota == orules_ref[0:1, :]).astype(jnp.float32)
    oh1 = (iota == orules_ref[1:2, :]).astype(jnp.float32)
    row0 = jnp.dot(oc, oh0, preferred_element_type=jnp.float32)[2:3, :]
    row1 = jnp.dot(oc, oh1, preferred_element_type=jnp.float32)[2:3, :]
    ones = jnp.ones((1, rb), jnp.float32)
    zeros = jnp.zeros((5, rb), jnp.float32)
    owt = jnp.concatenate([row0, row1, ones, zeros], axis=0)

    acc_ref[...] += lax.dot_general(
        owt, wmin, (((1,), (1,)), ((), ())),
        preferred_element_type=jnp.float32)

    @pl.when(i == nb - 1)
    def _finish():
        acc_out_ref[...] = acc_ref[...]


def _tc_shard(xrep, aux, rules_p, orules_p, r_tc):
    nb = r_tc // _RB
    return pl.pallas_call(
        functools.partial(_tc_shard_body, 6),
        grid=(nb,),
        in_specs=[
            pl.BlockSpec((_B, _LANES), lambda i: (0, 0)),
            pl.BlockSpec((8, _LANES), lambda i: (0, 0)),
            pl.BlockSpec((8, _RB), lambda i: (0, i)),
            pl.BlockSpec((8, _RB), lambda i: (0, i)),
        ],
        out_specs=pl.BlockSpec((8, _B), lambda i: (0, 0)),
        out_shape=jax.ShapeDtypeStruct((8, _B), jnp.float32),
        scratch_shapes=[pltpu.VMEM((8, _B), jnp.float32)],
    )(xrep, aux, rules_p, orules_p)


def _combine_body(acc_tc_ref, acc_sc_ref, out_ref):
    part = jnp.sum(acc_sc_ref[...][:, 0:3, :], axis=0)
    s3 = acc_tc_ref[0:3, :] + part
    den = jnp.maximum(s3[2:3, :], 1e-12)
    res = s3[0:2, :] / den
    out_ref[...] = jnp.concatenate([res, jnp.zeros((6, _B), jnp.float32)], 0)


def _combine(acc_tc, acc_sc):
    return pl.pallas_call(
        _combine_body,
        out_shape=jax.ShapeDtypeStruct((8, _B), jnp.float32),
    )(acc_tc, acc_sc)


def kernel(x, centers, sigmas, out_centers, input_rules, output_rules):
    b, nvar = x.shape
    m = centers.shape[1]
    f = nvar * m
    r = input_rules.shape[0]
    r_tc = r - _R_SC
    rpt = _R_SC // _NW

    xrep = jnp.broadcast_to(x[:, :, None], (b, nvar, m)).reshape(b, f)
    xrep = jnp.pad(xrep, ((0, 0), (0, _LANES - f)))
    aux = jnp.zeros((8, _LANES), jnp.float32)
    aux = aux.at[0, :f].set(centers.reshape(f))
    aux = aux.at[1, :].set(
        jnp.pad(sigmas.reshape(f), (0, _LANES - f), constant_values=1.0))
    aux = aux.at[2, :out_centers.shape[0]].set(out_centers)

    rules_p = jnp.zeros((8, r_tc), jnp.int32).at[:nvar, :].set(
        input_rules[:r_tc].T)
    orules_p = jnp.zeros((8, r_tc), jnp.int32).at[:2, :].set(
        output_rules[:r_tc].T)

    rules_sc = jnp.zeros((_NW, rpt, 8), jnp.int32)
    rules_sc = rules_sc.at[:, :, :nvar].set(
        input_rules[r_tc:].reshape(_NW, rpt, nvar))
    rules_sc = rules_sc.at[:, :, 6:8].set(
        output_rules[r_tc:].reshape(_NW, rpt, 2).astype(jnp.int32))
    rules_sc = rules_sc.reshape(_NW, rpt * 8)
    oc_pad = jnp.zeros((40,), jnp.float32).at[:out_centers.shape[0]].set(
        out_centers)

    x_flat = jnp.zeros((8, b), jnp.float32).at[:nvar, :].set(x.T).reshape(-1)
    cs_flat = jnp.concatenate([
        jnp.pad(centers.reshape(f), (0, _F - f)),
        jnp.pad(sigmas.reshape(f), (0, _F - f), constant_values=1.0),
    ])

    acc_tc = _tc_shard(xrep, aux, rules_p, orules_p, r_tc)
    acc_sc = _sc_shard(x_flat, cs_flat, rules_sc, oc_pad)
    out_t = _combine(acc_tc, acc_sc.reshape(_NW, 8, _B))
    return out_t[:2, :].T


# speedup vs baseline: 2.0178x; 2.0178x over previous
import functools

import jax
import jax.numpy as jnp
from jax import lax
from jax.experimental import pallas as pl
from jax.experimental.pallas import tpu as pltpu

_LANES = 128
_RB = 4096


def _anfis_body(nvar, xrep_ref, aux_ref, rules_ref, orules_ref, out_ref,
                acc_ref):
    i = pl.program_id(0)
    nb = pl.num_programs(0)
    rb = rules_ref.shape[1]

    @pl.when(i == 0)
    def _init():
        acc_ref[...] = jnp.zeros_like(acc_ref)

    c = aux_ref[0:1, :]
    s = aux_ref[1:2, :]
    d = xrep_ref[...] - c
    fuzz = jnp.exp(-(d * d) / (2.0 * s * s))

    iota = lax.broadcasted_iota(jnp.int32, (_LANES, rb), 0)

    fuzz_h = fuzz.astype(jnp.bfloat16)
    wmin = None
    for v in range(nvar):
        oh = (iota == rules_ref[v:v + 1, :]).astype(jnp.bfloat16)
        wv = jnp.dot(fuzz_h, oh, preferred_element_type=jnp.float32)
        wmin = wv if wmin is None else jnp.minimum(wmin, wv)

    oc = aux_ref[...]
    oh0 = (iota == orules_ref[0:1, :]).astype(jnp.float32)
    oh1 = (iota == orules_ref[1:2, :]).astype(jnp.float32)
    row0 = jnp.dot(oc, oh0, preferred_element_type=jnp.float32)[2:3, :]
    row1 = jnp.dot(oc, oh1, preferred_element_type=jnp.float32)[2:3, :]
    ones = jnp.ones((1, rb), jnp.float32)
    zeros = jnp.zeros((5, rb), jnp.float32)
    owt = jnp.concatenate([row0, row1, ones, zeros], axis=0)

    acc_ref[...] += lax.dot_general(
        wmin, owt, (((1,), (1,)), ((), ())),
        preferred_element_type=jnp.float32)

    @pl.when(i == nb - 1)
    def _finish():
        acc = acc_ref[...]
        den = jnp.maximum(acc[:, 2:3], 1e-12)
        out_ref[...] = acc[:, 0:2] / den


def kernel(x, centers, sigmas, out_centers, input_rules, output_rules):
    b, nvar = x.shape
    m = centers.shape[1]
    f = nvar * m
    r = input_rules.shape[0]
    nb = r // _RB

    xrep = jnp.broadcast_to(x[:, :, None], (b, nvar, m)).reshape(b, f)
    xrep = jnp.pad(xrep, ((0, 0), (0, _LANES - f)))
    aux = jnp.zeros((8, _LANES), jnp.float32)
    aux = aux.at[0, :f].set(centers.reshape(f))
    aux = aux.at[1, :].set(
        jnp.pad(sigmas.reshape(f), (0, _LANES - f), constant_values=1.0))
    aux = aux.at[2, :out_centers.shape[0]].set(out_centers)
    rules_p = jnp.zeros((8, r), jnp.int32).at[:nvar, :].set(input_rules.T)
    orules_p = jnp.zeros((8, r), jnp.int32).at[:2, :].set(output_rules.T)

    return pl.pallas_call(
        functools.partial(_anfis_body, nvar),
        grid=(nb,),
        in_specs=[
            pl.BlockSpec((b, _LANES), lambda i: (0, 0)),
            pl.BlockSpec((8, _LANES), lambda i: (0, 0)),
            pl.BlockSpec((8, _RB), lambda i: (0, i)),
            pl.BlockSpec((8, _RB), lambda i: (0, i)),
        ],
        out_specs=pl.BlockSpec((b, 2), lambda i: (0, 0)),
        out_shape=jax.ShapeDtypeStruct((b, 2), jnp.float32),
        scratch_shapes=[pltpu.VMEM((b, 8), jnp.float32)],
    )(xrep, aux, rules_p, orules_p)
